# Initial kernel scaffold; baseline (speedup 1.0000x reference)
import functools

import jax
import jax.numpy as jnp
from jax import lax
from jax.experimental import pallas as pl
from jax.experimental.pallas import tpu as pltpu
from jax.experimental.pallas import tpu_sc as plsc

_COMMIT = 0.25
_N = 16384
_K = 8192
_D = 256
_BLK = 256
_GRID = _N // _BLK


def _layernorm(x, g, b):
    m = jnp.mean(x, axis=-1, keepdims=True)
    v = jnp.var(x, axis=-1, keepdims=True)
    return (x - m) / jnp.sqrt(v + 1e-5) * g + b


def _enc_body(x_ref, w0_ref, b0_ref, g0_ref, be0_ref, w1_ref, b1_ref, g1_ref,
              be1_ref, w2_ref, b2_ref, cbt_ref, ze_ref, idx_ref):
    h = x_ref[...] @ w0_ref[...] + b0_ref[...]
    h = jax.nn.gelu(_layernorm(h, g0_ref[...], be0_ref[...]), approximate=False)
    h = h @ w1_ref[...] + b1_ref[...]
    h = jax.nn.gelu(_layernorm(h, g1_ref[...], be1_ref[...]), approximate=False)
    z = h @ w2_ref[...] + b2_ref[...]
    ze_ref[...] = z

    cbt = cbt_ref[...]
    scores = z @ cbt
    znorm = jnp.sum(z * z, axis=1, keepdims=True)
    cnorm = jnp.sum(cbt * cbt, axis=0)
    dist = znorm - 2.0 * scores + cnorm[None, :]
    mins = jnp.min(dist, axis=1, keepdims=True)
    cols = lax.broadcasted_iota(jnp.int32, dist.shape, 1)
    idx = jnp.min(jnp.where(dist == mins, cols, jnp.int32(_K)), axis=1)
    idx_ref[...] = idx[:, None]


def _dec_body(zq_ref, ze_ref, x_ref, w0_ref, b0_ref, g0_ref, be0_ref, w1_ref,
              b1_ref, g1_ref, be1_ref, w2_ref, b2_ref, xr_ref, zqst_ref,
              rec_ref, vq_ref):
    zq = zq_ref[...]
    ze = ze_ref[...]
    zqst = ze + (zq - ze)
    zqst_ref[...] = zqst
    h = zqst @ w0_ref[...] + b0_ref[...]
    h = jax.nn.gelu(_layernorm(h, g0_ref[...], be0_ref[...]), approximate=False)
    h = h @ w1_ref[...] + b1_ref[...]
    h = jax.nn.gelu(_layernorm(h, g1_ref[...], be1_ref[...]), approximate=False)
    xr = h @ w2_ref[...] + b2_ref[...]
    xr_ref[...] = xr

    @pl.when(pl.program_id(0) == 0)
    def _():
        rec_ref[...] = jnp.zeros_like(rec_ref)
        vq_ref[...] = jnp.zeros_like(vq_ref)

    rec_ref[...] += jnp.sum((xr - x_ref[...]) ** 2).reshape(1, 1)
    vq_ref[...] += jnp.sum((zq - ze) ** 2).reshape(1, 1)


def _full(shape):
    zeros = (0,) * len(shape)
    return pl.BlockSpec(shape, lambda i: zeros)


def _encode_quantize(x, w0, b0, g0, be0, w1, b1, g1, be1, w2, b2, cbt):
    return pl.pallas_call(
        _enc_body,
        grid=(_GRID,),
        in_specs=[
            pl.BlockSpec((_BLK, 768), lambda i: (i, 0)),
            _full((768, 512)), _full((512,)), _full((512,)), _full((512,)),
            _full((512, 384)), _full((384,)), _full((384,)), _full((384,)),
            _full((384, 256)), _full((256,)),
            _full((_D, _K)),
        ],
        out_specs=[
            pl.BlockSpec((_BLK, _D), lambda i: (i, 0)),
            pl.BlockSpec((_BLK, 1), lambda i: (i, 0)),
        ],
        out_shape=[
            jax.ShapeDtypeStruct((_N, _D), jnp.float32),
            jax.ShapeDtypeStruct((_N, 1), jnp.int32),
        ],
    )(x, w0, b0, g0, be0, w1, b1, g1, be1, w2, b2, cbt)


def _sc_gather(codebook, idx):
    info = plsc.get_sparse_core_info()
    nw = info.num_cores * info.num_subcores
    b_per_w = _N // nw
    chunk = 128
    nchunk = b_per_w // chunk
    mesh = plsc.VectorSubcoreMesh(core_axis_name="c", subcore_axis_name="s")

    @functools.partial(
        pl.kernel, mesh=mesh,
        out_type=jax.ShapeDtypeStruct((_N, _D), jnp.float32),
        scratch_types=[
            pltpu.VMEM((chunk,), jnp.int32),
            pltpu.VMEM((chunk, _D), jnp.float32),
            pltpu.SemaphoreType.DMA,
        ],
    )
    def gather(table_hbm, idx_hbm, out_hbm, idx_v, rows_v, sem):
        wid = lax.axis_index("s") * info.num_cores + lax.axis_index("c")
        for j in range(nchunk):
            base = wid * b_per_w + j * chunk
            pltpu.sync_copy(idx_hbm.at[pl.ds(base, chunk)], idx_v)
            pltpu.async_copy(table_hbm.at[idx_v], rows_v, sem).wait()
            pltpu.sync_copy(rows_v, out_hbm.at[pl.ds(base, chunk)])

    return gather(codebook, idx)


def _decode(zq, ze, x, w0, b0, g0, be0, w1, b1, g1, be1, w2, b2):
    return pl.pallas_call(
        _dec_body,
        grid=(_GRID,),
        in_specs=[
            pl.BlockSpec((_BLK, _D), lambda i: (i, 0)),
            pl.BlockSpec((_BLK, _D), lambda i: (i, 0)),
            pl.BlockSpec((_BLK, 768), lambda i: (i, 0)),
            _full((256, 384)), _full((384,)), _full((384,)), _full((384,)),
            _full((384, 512)), _full((512,)), _full((512,)), _full((512,)),
            _full((512, 768)), _full((768,)),
        ],
        out_specs=[
            pl.BlockSpec((_BLK, 768), lambda i: (i, 0)),
            pl.BlockSpec((_BLK, _D), lambda i: (i, 0)),
            pl.BlockSpec((1, 1), lambda i: (0, 0)),
            pl.BlockSpec((1, 1), lambda i: (0, 0)),
        ],
        out_shape=[
            jax.ShapeDtypeStruct((_N, 768), jnp.float32),
            jax.ShapeDtypeStruct((_N, _D), jnp.float32),
            jax.ShapeDtypeStruct((1, 1), jnp.float32),
            jax.ShapeDtypeStruct((1, 1), jnp.float32),
        ],
    )(zq, ze, x, w0, b0, g0, be0, w1, b1, g1, be1, w2, b2)


def kernel(x, enc_W0, enc_b0, enc_g0, enc_be0, enc_W1, enc_b1, enc_g1, enc_be1,
           enc_W2, enc_b2, codebook, dec_W0, dec_b0, dec_g0, dec_be0, dec_W1,
           dec_b1, dec_g1, dec_be1, dec_W2, dec_b2):
    ze, idx2 = _encode_quantize(x, enc_W0, enc_b0, enc_g0, enc_be0, enc_W1,
                                enc_b1, enc_g1, enc_be1, enc_W2, enc_b2,
                                codebook.T)
    idx = idx2.reshape(_N)
    zq = _sc_gather(codebook, idx)
    xr, zqst, rec_sum, vq_sum = _decode(zq, ze, x, dec_W0, dec_b0, dec_g0,
                                        dec_be0, dec_W1, dec_b1, dec_g1,
                                        dec_be1, dec_W2, dec_b2)
    recon_loss = (rec_sum[0, 0] / (_N * 768)).reshape(())
    vq_loss = (_COMMIT * (vq_sum[0, 0] / (_N * _D))).reshape(())
    total_loss = recon_loss + vq_loss
    return (xr, total_loss, recon_loss, vq_loss, idx, ze, zqst)


# speedup vs baseline: 1.2957x; 1.2957x over previous
import functools

import jax
import jax.numpy as jnp
from jax import lax
from jax.experimental import pallas as pl
from jax.experimental.pallas import tpu as pltpu
from jax.experimental.pallas import tpu_sc as plsc

_COMMIT = 0.25
_N = 16384
_K = 8192
_D = 256
_BLK = 256
_GRID = _N // _BLK


def _mm(a, b):
    return jax.lax.dot(a, b, precision=None)


def _gelu(x):
    return 0.5 * x * (1.0 + lax.erf(x * 0.7071067811865476))


def _layernorm(x, g, b):
    m = jnp.mean(x, axis=-1, keepdims=True)
    v = jnp.var(x, axis=-1, keepdims=True)
    return (x - m) / jnp.sqrt(v + 1e-5) * g + b


def _enc_body(x_ref, w0_ref, b0_ref, g0_ref, be0_ref, w1_ref, b1_ref, g1_ref,
              be1_ref, w2_ref, b2_ref, cbt_ref, ze_ref, idx_ref):
    h = _mm(x_ref[...], w0_ref[...]) + b0_ref[...]
    h = _gelu(_layernorm(h, g0_ref[...], be0_ref[...]))
    h = _mm(h, w1_ref[...]) + b1_ref[...]
    h = _gelu(_layernorm(h, g1_ref[...], be1_ref[...]))
    z = _mm(h, w2_ref[...]) + b2_ref[...]
    ze_ref[...] = z

    cbt = cbt_ref[...]
    scores = _mm(z, cbt)
    znorm = jnp.sum(z * z, axis=1, keepdims=True)
    cnorm = jnp.sum(cbt * cbt, axis=0)
    dist = znorm - 2.0 * scores + cnorm[None, :]
    mins = jnp.min(dist, axis=1, keepdims=True)
    cols = lax.broadcasted_iota(jnp.int32, dist.shape, 1)
    idx = jnp.min(jnp.where(dist == mins, cols, jnp.int32(_K)), axis=1)
    idx_ref[...] = idx[:, None]


def _dec_body(zq_ref, ze_ref, x_ref, w0_ref, b0_ref, g0_ref, be0_ref, w1_ref,
              b1_ref, g1_ref, be1_ref, w2_ref, b2_ref, xr_ref, zqst_ref,
              rec_ref, vq_ref):
    zq = zq_ref[...]
    ze = ze_ref[...]
    zqst = ze + (zq - ze)
    zqst_ref[...] = zqst
    h = _mm(zqst, w0_ref[...]) + b0_ref[...]
    h = _gelu(_layernorm(h, g0_ref[...], be0_ref[...]))
    h = _mm(h, w1_ref[...]) + b1_ref[...]
    h = _gelu(_layernorm(h, g1_ref[...], be1_ref[...]))
    xr = _mm(h, w2_ref[...]) + b2_ref[...]
    xr_ref[...] = xr

    @pl.when(pl.program_id(0) == 0)
    def _():
        rec_ref[...] = jnp.zeros_like(rec_ref)
        vq_ref[...] = jnp.zeros_like(vq_ref)

    rec_ref[...] += jnp.sum((xr - x_ref[...]) ** 2).reshape(1, 1)
    vq_ref[...] += jnp.sum((zq - ze) ** 2).reshape(1, 1)


def _full(shape):
    zeros = (0,) * len(shape)
    return pl.BlockSpec(shape, lambda i: zeros)


def _encode_quantize(x, w0, b0, g0, be0, w1, b1, g1, be1, w2, b2, cbt):
    return pl.pallas_call(
        _enc_body,
        grid=(_GRID,),
        in_specs=[
            pl.BlockSpec((_BLK, 768), lambda i: (i, 0)),
            _full((768, 512)), _full((512,)), _full((512,)), _full((512,)),
            _full((512, 384)), _full((384,)), _full((384,)), _full((384,)),
            _full((384, 256)), _full((256,)),
            _full((_D, _K)),
        ],
        out_specs=[
            pl.BlockSpec((_BLK, _D), lambda i: (i, 0)),
            pl.BlockSpec((_BLK, 1), lambda i: (i, 0)),
        ],
        out_shape=[
            jax.ShapeDtypeStruct((_N, _D), jnp.float32),
            jax.ShapeDtypeStruct((_N, 1), jnp.int32),
        ],
    )(x, w0, b0, g0, be0, w1, b1, g1, be1, w2, b2, cbt)


def _sc_gather(codebook, idx):
    info = plsc.get_sparse_core_info()
    nw = info.num_cores * info.num_subcores
    b_per_w = _N // nw
    chunk = 128
    nchunk = b_per_w // chunk
    mesh = plsc.VectorSubcoreMesh(core_axis_name="c", subcore_axis_name="s")

    @functools.partial(
        pl.kernel, mesh=mesh,
        out_type=jax.ShapeDtypeStruct((_N, _D), jnp.float32),
        scratch_types=[
            pltpu.VMEM((chunk,), jnp.int32),
            pltpu.VMEM((chunk, _D), jnp.float32),
            pltpu.SemaphoreType.DMA,
        ],
    )
    def gather(table_hbm, idx_hbm, out_hbm, idx_v, rows_v, sem):
        wid = lax.axis_index("s") * info.num_cores + lax.axis_index("c")
        for j in range(nchunk):
            base = wid * b_per_w + j * chunk
            pltpu.sync_copy(idx_hbm.at[pl.ds(base, chunk)], idx_v)
            pltpu.async_copy(table_hbm.at[idx_v], rows_v, sem).wait()
            pltpu.sync_copy(rows_v, out_hbm.at[pl.ds(base, chunk)])

    return gather(codebook, idx)


def _decode(zq, ze, x, w0, b0, g0, be0, w1, b1, g1, be1, w2, b2):
    return pl.pallas_call(
        _dec_body,
        grid=(_GRID,),
        in_specs=[
            pl.BlockSpec((_BLK, _D), lambda i: (i, 0)),
            pl.BlockSpec((_BLK, _D), lambda i: (i, 0)),
            pl.BlockSpec((_BLK, 768), lambda i: (i, 0)),
            _full((256, 384)), _full((384,)), _full((384,)), _full((384,)),
            _full((384, 512)), _full((512,)), _full((512,)), _full((512,)),
            _full((512, 768)), _full((768,)),
        ],
        out_specs=[
            pl.BlockSpec((_BLK, 768), lambda i: (i, 0)),
            pl.BlockSpec((_BLK, _D), lambda i: (i, 0)),
            pl.BlockSpec((1, 1), lambda i: (0, 0)),
            pl.BlockSpec((1, 1), lambda i: (0, 0)),
        ],
        out_shape=[
            jax.ShapeDtypeStruct((_N, 768), jnp.float32),
            jax.ShapeDtypeStruct((_N, _D), jnp.float32),
            jax.ShapeDtypeStruct((1, 1), jnp.float32),
            jax.ShapeDtypeStruct((1, 1), jnp.float32),
        ],
    )(zq, ze, x, w0, b0, g0, be0, w1, b1, g1, be1, w2, b2)


def kernel(x, enc_W0, enc_b0, enc_g0, enc_be0, enc_W1, enc_b1, enc_g1, enc_be1,
           enc_W2, enc_b2, codebook, dec_W0, dec_b0, dec_g0, dec_be0, dec_W1,
           dec_b1, dec_g1, dec_be1, dec_W2, dec_b2):
    ze, idx2 = _encode_quantize(x, enc_W0, enc_b0, enc_g0, enc_be0, enc_W1,
                                enc_b1, enc_g1, enc_be1, enc_W2, enc_b2,
                                codebook.T)
    idx = idx2.reshape(_N)
    zq = _sc_gather(codebook, idx)
    xr, zqst, rec_sum, vq_sum = _decode(zq, ze, x, dec_W0, dec_b0, dec_g0,
                                        dec_be0, dec_W1, dec_b1, dec_g1,
                                        dec_be1, dec_W2, dec_b2)
    recon_loss = (rec_sum[0, 0] / (_N * 768)).reshape(())
    vq_loss = (_COMMIT * (vq_sum[0, 0] / (_N * _D))).reshape(())
    total_loss = recon_loss + vq_loss
    return (xr, total_loss, recon_loss, vq_loss, idx, ze, zqst)
